# baseline (speedup 1.0000x reference)
import jax
import jax.numpy as jnp
from jax.experimental import pallas as pl
from jax.experimental.pallas import tpu as pltpu


def kernel(x, offsets):
    tokens, d = x.shape
    nseg = offsets.shape[0] - 1
    sub = d // 128

    def _pool(offs_ref, x_ref, out_ref):
        out_ref[...] = x_ref[...]

    grid_spec = pltpu.PrefetchScalarGridSpec(
        num_scalar_prefetch=1,
        grid=(nseg,),
        in_specs=[
            pl.BlockSpec((1, sub, 128), lambda i, offs: (offs[i + 1] - 1, 0, 0))
        ],
        out_specs=pl.BlockSpec((1, sub, 128), lambda i, offs: (i, 0, 0)),
    )

    out = pl.pallas_call(
        _pool,
        grid_spec=grid_spec,
        out_shape=jax.ShapeDtypeStruct((nseg, sub, 128), x.dtype),
    )(offsets.astype(jnp.int32), x.reshape(tokens, sub, 128))
    return out.reshape(nseg, d)


# speedup vs baseline: 21.2812x; 21.2812x over previous
import jax
import jax.numpy as jnp
from jax.experimental import pallas as pl
from jax.experimental.pallas import tpu as pltpu


def kernel(x, offsets):
    tokens, d = x.shape
    nseg = offsets.shape[0] - 1

    def _pool(offs_ref, x_ref, out_ref, buf, in_sems, out_sems):
        fetches = []
        for i in range(nseg):
            row = offs_ref[i + 1] - 1
            fetches.append(
                pltpu.make_async_copy(
                    x_ref.at[pl.ds(row, 1)], buf.at[pl.ds(i, 1)], in_sems.at[i]
                )
            )
        for c in fetches:
            c.start()
        writebacks = []
        for i in range(nseg):
            fetches[i].wait()
            wb = pltpu.make_async_copy(
                buf.at[pl.ds(i, 1)], out_ref.at[pl.ds(i, 1)], out_sems.at[i]
            )
            wb.start()
            writebacks.append(wb)
        for wb in writebacks:
            wb.wait()

    grid_spec = pltpu.PrefetchScalarGridSpec(
        num_scalar_prefetch=1,
        grid=(1,),
        in_specs=[pl.BlockSpec(memory_space=pl.ANY)],
        out_specs=pl.BlockSpec(memory_space=pl.ANY),
        scratch_shapes=[
            pltpu.VMEM((nseg, d), jnp.float32),
            pltpu.SemaphoreType.DMA((nseg,)),
            pltpu.SemaphoreType.DMA((nseg,)),
        ],
    )

    return pl.pallas_call(
        _pool,
        grid_spec=grid_spec,
        out_shape=jax.ShapeDtypeStruct((nseg, d), x.dtype),
    )(offsets.astype(jnp.int32), x)


# speedup vs baseline: 22.9740x; 1.0795x over previous
import jax
import jax.numpy as jnp
from jax.experimental import pallas as pl
from jax.experimental.pallas import tpu as pltpu


def kernel(x, offsets):
    tokens, d = x.shape
    nseg = offsets.shape[0] - 1

    def _pool(offs_ref, x_ref, out_ref, sem):
        copies = []
        for i in range(nseg):
            row = offs_ref[i + 1] - 1
            copies.append(
                pltpu.make_async_copy(
                    x_ref.at[pl.ds(row, 1)], out_ref.at[pl.ds(i, 1)], sem
                )
            )
        for c in copies:
            c.start()
        for c in copies:
            c.wait()

    grid_spec = pltpu.PrefetchScalarGridSpec(
        num_scalar_prefetch=1,
        grid=(1,),
        in_specs=[pl.BlockSpec(memory_space=pl.ANY)],
        out_specs=pl.BlockSpec((nseg, d), lambda i, offs: (0, 0)),
        scratch_shapes=[pltpu.SemaphoreType.DMA],
    )

    return pl.pallas_call(
        _pool,
        grid_spec=grid_spec,
        out_shape=jax.ShapeDtypeStruct((nseg, d), x.dtype),
    )(offsets.astype(jnp.int32), x)
